# baseline (speedup 1.0000x reference)
import functools

import jax
import jax.numpy as jnp
from jax import lax
from jax.experimental import pallas as pl
from jax.experimental.pallas import tpu as pltpu
from jax.experimental.pallas import tpu_sc as plsc

H = 768
FF = 3072
E = 8
K = 2
T = 2048
TM = 128
NT = (T * K + E * (TM - 1) + TM - 1) // TM
P = NT * TM
FC = 768
NW = 32
TPW = T // NW
GW = 128


def _router_body(x_ref, wr_ref, logits_ref, w0x_ref, w1x_ref, slot01_ref,
                 eid_ref, act_ref):
    x = x_ref[...]
    logits = lax.dot_general(
        x, wr_ref[...], (((1,), (1,)), ((), ())),
        preferred_element_type=jnp.float32)
    logits_ref[...] = logits
    m = jnp.max(logits, axis=-1, keepdims=True)
    p = jnp.exp(logits - m)
    rw = p / jnp.sum(p, axis=-1, keepdims=True)
    rank = jnp.zeros((T, E), jnp.int32)
    col = lax.broadcasted_iota(jnp.int32, (T, E), 1)
    for j in range(E):
        lj = logits[:, j:j + 1]
        rank = rank + jnp.where(
            (lj > logits) | ((lj == logits) & (j < col)), 1, 0)
    top0 = rank == 0
    top1 = rank == 1
    w0 = jnp.sum(jnp.where(top0, rw, 0.0), axis=1, keepdims=True)
    w1 = jnp.sum(jnp.where(top1, rw, 0.0), axis=1, keepdims=True)
    w0x_ref[...] = jnp.broadcast_to(w0, (T, GW))
    w1x_ref[...] = jnp.broadcast_to(w1, (T, GW))

    oh01 = (top0 | top1).astype(jnp.bfloat16)
    TB = 256
    rB = lax.broadcasted_iota(jnp.int32, (TB, TB), 0)
    cB = lax.broadcasted_iota(jnp.int32, (TB, TB), 1)
    trib = (cB < rB).astype(jnp.bfloat16)
    rows = []
    acc = jnp.zeros((1, E), jnp.float32)
    for b in range(T // TB):
        ohb = oh01[b * TB:(b + 1) * TB, :]
        pcs = lax.dot_general(trib, ohb, (((1,), (0,)), ((), ())),
                              preferred_element_type=jnp.float32)
        rows.append(pcs + acc)
        acc = acc + jnp.sum(ohb.astype(jnp.float32), axis=0, keepdims=True)
    csum = jnp.concatenate(rows, axis=0)
    gs = acc
    nt = jnp.floor((gs + (TM - 1)) * (1.0 / TM))
    r8 = lax.broadcasted_iota(jnp.int32, (E, E), 0)
    c8 = lax.broadcasted_iota(jnp.int32, (E, E), 1)
    m8 = (r8 < c8).astype(jnp.float32)
    pst = lax.dot_general(nt, m8, (((1,), (0,)), ((), ())),
                          precision=lax.Precision.HIGHEST,
                          preferred_element_type=jnp.float32)
    pstart_rows = pst * TM
    base = jnp.broadcast_to(pstart_rows, (T, E)) + csum
    s0 = jnp.sum(jnp.where(top0, base, 0.0), axis=1, keepdims=True)
    s1 = jnp.sum(jnp.where(top1, base, 0.0), axis=1, keepdims=True)
    slot01_ref[...] = jnp.concatenate([s0, s1], axis=1).astype(jnp.int32)

    ti = lax.broadcasted_iota(jnp.int32, (1, NT), 1)
    pst_i = pst.astype(jnp.int32)
    nt_i = nt.astype(jnp.int32)
    eid = jnp.full((1, NT), E - 1, jnp.int32)
    for e in range(E - 1):
        st = pst_i[0:1, e:e + 1]
        en = st + nt_i[0:1, e:e + 1]
        eid = jnp.where((ti >= st) & (ti < en), e, eid)
    total = jnp.sum(nt_i, axis=1, keepdims=True)
    act_ref[...] = (ti < total).astype(jnp.int32)
    eid_ref[...] = eid


@jax.jit
def _router(x, Wr):
    return pl.pallas_call(
        _router_body,
        out_shape=[
            jax.ShapeDtypeStruct((T, E), jnp.float32),
            jax.ShapeDtypeStruct((T, GW), jnp.float32),
            jax.ShapeDtypeStruct((T, GW), jnp.float32),
            jax.ShapeDtypeStruct((T, K), jnp.int32),
            jax.ShapeDtypeStruct((1, NT), jnp.int32),
            jax.ShapeDtypeStruct((1, NT), jnp.int32),
        ],
    )(x, Wr)


def _dispatch_kernel(x_hbm, sl_hbm, w0x_hbm, w1x_hbm, xs_hbm, gv_hbm,
                     xrows_v, w0_v, w1_v, idx0_v, idx1_v, sem):
    wid = lax.axis_index("s") * 2 + lax.axis_index("c")
    base = wid * TPW
    pltpu.sync_copy(x_hbm.at[pl.ds(base, TPW)], xrows_v)
    pltpu.sync_copy(w0x_hbm.at[pl.ds(base, TPW)], w0_v)
    pltpu.sync_copy(w1x_hbm.at[pl.ds(base, TPW)], w1_v)
    pltpu.sync_copy(sl_hbm.at[0, pl.ds(base, TPW)], idx0_v)
    pltpu.sync_copy(sl_hbm.at[1, pl.ds(base, TPW)], idx1_v)
    c0 = pltpu.make_async_copy(xrows_v, xs_hbm.at[idx0_v], sem)
    c1 = pltpu.make_async_copy(xrows_v, xs_hbm.at[idx1_v], sem)
    g0 = pltpu.make_async_copy(w0_v, gv_hbm.at[idx0_v], sem)
    g1 = pltpu.make_async_copy(w1_v, gv_hbm.at[idx1_v], sem)
    c0.start()
    c1.start()
    g0.start()
    g1.start()
    c0.wait()
    c1.wait()
    g0.wait()
    g1.wait()


@jax.jit
def _dispatch(x, slT, w0x, w1x):
    mesh = plsc.VectorSubcoreMesh(core_axis_name="c", subcore_axis_name="s")
    f = functools.partial(
        pl.kernel,
        mesh=mesh,
        out_type=[
            jax.ShapeDtypeStruct((P, H), jnp.float32),
            jax.ShapeDtypeStruct((P, GW), jnp.float32),
        ],
        scratch_types=[
            pltpu.VMEM((TPW, H), jnp.float32),
            pltpu.VMEM((TPW, GW), jnp.float32),
            pltpu.VMEM((TPW, GW), jnp.float32),
            pltpu.VMEM((TPW,), jnp.int32),
            pltpu.VMEM((TPW,), jnp.int32),
            pltpu.SemaphoreType.DMA,
        ],
    )(_dispatch_kernel)
    return f(x, slT, w0x, w1x)


def _gmm_body(eid_ref, act_ref, x_ref, w1_ref, b1_ref, w2_ref, b2_ref,
              gv_ref, hb_ref, out_ref):
    i = pl.program_id(0)

    @pl.when(act_ref[i] == 1)
    def _compute():
        xt = x_ref[...].astype(jnp.bfloat16)
        y = jnp.zeros((TM, H), jnp.float32)
        for fc in range(FF // FC):
            w1c = w1_ref[0, pl.ds(fc * FC, FC), :].astype(jnp.bfloat16)
            h = lax.dot_general(
                xt, w1c, (((1,), (1,)), ((), ())),
                preferred_element_type=jnp.float32)
            h = h + b1_ref[0, :, pl.ds(fc * FC, FC)]
            h = (0.5 * h * (1.0 + lax.erf(h * 0.7071067811865476))).astype(
                jnp.bfloat16)
            w2c = w2_ref[0, :, pl.ds(fc * FC, FC)].astype(jnp.bfloat16)
            y = y + lax.dot_general(
                h, w2c, (((1,), (1,)), ((), ())),
                preferred_element_type=jnp.float32)
        g = gv_ref[:, 0:1]
        out_ref[...] = (y + b2_ref[0]) * g + hb_ref[...]


@jax.jit
def _gmm(eid, act, xs, W1, b1, W2, b2, gv, halfbias):
    grid_spec = pltpu.PrefetchScalarGridSpec(
        num_scalar_prefetch=2,
        grid=(NT,),
        in_specs=[
            pl.BlockSpec((TM, H), lambda i, eid, act: (act[i] * i, 0)),
            pl.BlockSpec((1, FF, H), lambda i, eid, act: (eid[i], 0, 0)),
            pl.BlockSpec((1, 1, FF), lambda i, eid, act: (eid[i], 0, 0)),
            pl.BlockSpec((1, H, FF), lambda i, eid, act: (eid[i], 0, 0)),
            pl.BlockSpec((1, 1, H), lambda i, eid, act: (eid[i], 0, 0)),
            pl.BlockSpec((TM, GW), lambda i, eid, act: (act[i] * i, 0)),
            pl.BlockSpec((1, H), lambda i, eid, act: (0, 0)),
        ],
        out_specs=pl.BlockSpec(
            (TM, H),
            lambda i, eid, act: (act[i] * i + (1 - act[i]) * (NT - 1), 0)),
    )
    return pl.pallas_call(
        _gmm_body,
        grid_spec=grid_spec,
        out_shape=jax.ShapeDtypeStruct((P, H), jnp.float32),
    )(eid, act, xs, W1, b1.reshape(E, 1, FF), W2, b2.reshape(E, 1, H), gv,
      halfbias)


def _combine_kernel(yg_hbm, sl_hbm, out_hbm, buf0_v, buf1_v,
                    idx0_v, idx1_v, sem):
    wid = lax.axis_index("s") * 2 + lax.axis_index("c")
    base = wid * TPW
    pltpu.sync_copy(sl_hbm.at[0, pl.ds(base, TPW)], idx0_v)
    pltpu.sync_copy(sl_hbm.at[1, pl.ds(base, TPW)], idx1_v)
    c0 = pltpu.make_async_copy(yg_hbm.at[idx0_v], buf0_v, sem)
    c1 = pltpu.make_async_copy(yg_hbm.at[idx1_v], buf1_v, sem)
    c0.start()
    c1.start()
    c0.wait()
    c1.wait()

    def row(r, carry):
        for c in range(H // 16):
            sl = pl.ds(c * 16, 16)
            buf0_v[r, sl] = buf0_v[r, sl] + buf1_v[r, sl]
        return carry

    lax.fori_loop(0, TPW, row, 0)
    pltpu.sync_copy(buf0_v, out_hbm.at[pl.ds(base, TPW)])


@jax.jit
def _combine(yg, slT):
    mesh = plsc.VectorSubcoreMesh(core_axis_name="c", subcore_axis_name="s")
    f = functools.partial(
        pl.kernel,
        mesh=mesh,
        out_type=jax.ShapeDtypeStruct((T, H), jnp.float32),
        scratch_types=[
            pltpu.VMEM((TPW, H), jnp.float32),
            pltpu.VMEM((TPW, H), jnp.float32),
            pltpu.VMEM((TPW,), jnp.int32),
            pltpu.VMEM((TPW,), jnp.int32),
            pltpu.SemaphoreType.DMA,
        ],
    )(_combine_kernel)
    return f(yg, slT)


def kernel(hidden_states, Wr, W1, b1, W2, b2, bias):
    input_shape = hidden_states.shape
    x = hidden_states.reshape(-1, input_shape[-1])
    logits, w0x, w1x, slot01, eid, act = _router(x, Wr)
    slT = slot01.T
    xs, gv = _dispatch(x, slT, w0x, w1x)
    yg = _gmm(eid.reshape(NT), act.reshape(NT), xs, W1, b1, W2, b2, gv,
              0.5 * bias.reshape(1, H))
    out = _combine(yg, slT)
    return out.reshape(input_shape), logits


# speedup vs baseline: 1.3837x; 1.3837x over previous
import functools

import jax
import jax.numpy as jnp
from jax import lax
from jax.experimental import pallas as pl
from jax.experimental.pallas import tpu as pltpu
from jax.experimental.pallas import tpu_sc as plsc

H = 768
FF = 3072
E = 8
K = 2
T = 2048
TM = 256
NT = (T * K + E * (TM - 1) + TM - 1) // TM
P = NT * TM
FC = 768
NW = 32
TPW = T // NW
GW = 128


def _router_body(x_ref, wr_ref, logits_ref, w0x_ref, w1x_ref, slot01_ref,
                 eid_ref, act_ref):
    x = x_ref[...]
    logits = lax.dot_general(
        x, wr_ref[...], (((1,), (1,)), ((), ())),
        preferred_element_type=jnp.float32)
    logits_ref[...] = logits
    m = jnp.max(logits, axis=-1, keepdims=True)
    p = jnp.exp(logits - m)
    rw = p / jnp.sum(p, axis=-1, keepdims=True)
    rank = jnp.zeros((T, E), jnp.int32)
    col = lax.broadcasted_iota(jnp.int32, (T, E), 1)
    for j in range(E):
        lj = logits[:, j:j + 1]
        rank = rank + jnp.where(
            (lj > logits) | ((lj == logits) & (j < col)), 1, 0)
    top0 = rank == 0
    top1 = rank == 1
    w0 = jnp.sum(jnp.where(top0, rw, 0.0), axis=1, keepdims=True)
    w1 = jnp.sum(jnp.where(top1, rw, 0.0), axis=1, keepdims=True)
    w0x_ref[...] = jnp.broadcast_to(w0, (T, GW))
    w1x_ref[...] = jnp.broadcast_to(w1, (T, GW))

    oh01 = (top0 | top1).astype(jnp.bfloat16)
    TB = 256
    rB = lax.broadcasted_iota(jnp.int32, (TB, TB), 0)
    cB = lax.broadcasted_iota(jnp.int32, (TB, TB), 1)
    trib = (cB < rB).astype(jnp.bfloat16)
    rows = []
    acc = jnp.zeros((1, E), jnp.float32)
    for b in range(T // TB):
        ohb = oh01[b * TB:(b + 1) * TB, :]
        pcs = lax.dot_general(trib, ohb, (((1,), (0,)), ((), ())),
                              preferred_element_type=jnp.float32)
        rows.append(pcs + acc)
        acc = acc + jnp.sum(ohb.astype(jnp.float32), axis=0, keepdims=True)
    csum = jnp.concatenate(rows, axis=0)
    gs = acc
    nt = jnp.floor((gs + (TM - 1)) * (1.0 / TM))
    r8 = lax.broadcasted_iota(jnp.int32, (E, E), 0)
    c8 = lax.broadcasted_iota(jnp.int32, (E, E), 1)
    m8 = (r8 < c8).astype(jnp.float32)
    pst = lax.dot_general(nt, m8, (((1,), (0,)), ((), ())),
                          precision=lax.Precision.HIGHEST,
                          preferred_element_type=jnp.float32)
    pstart_rows = pst * TM
    base = jnp.broadcast_to(pstart_rows, (T, E)) + csum
    s0 = jnp.sum(jnp.where(top0, base, 0.0), axis=1, keepdims=True)
    s1 = jnp.sum(jnp.where(top1, base, 0.0), axis=1, keepdims=True)
    slot01_ref[...] = jnp.concatenate([s0, s1], axis=1).astype(jnp.int32)

    ti = lax.broadcasted_iota(jnp.int32, (1, NT), 1)
    pst_i = pst.astype(jnp.int32)
    nt_i = nt.astype(jnp.int32)
    eid = jnp.full((1, NT), E - 1, jnp.int32)
    for e in range(E - 1):
        st = pst_i[0:1, e:e + 1]
        en = st + nt_i[0:1, e:e + 1]
        eid = jnp.where((ti >= st) & (ti < en), e, eid)
    total = jnp.sum(nt_i, axis=1, keepdims=True)
    act_ref[...] = (ti < total).astype(jnp.int32)
    eid_ref[...] = eid


@jax.jit
def _router(x, Wr):
    return pl.pallas_call(
        _router_body,
        out_shape=[
            jax.ShapeDtypeStruct((T, E), jnp.float32),
            jax.ShapeDtypeStruct((T, GW), jnp.float32),
            jax.ShapeDtypeStruct((T, GW), jnp.float32),
            jax.ShapeDtypeStruct((T, K), jnp.int32),
            jax.ShapeDtypeStruct((1, NT), jnp.int32),
            jax.ShapeDtypeStruct((1, NT), jnp.int32),
        ],
    )(x, Wr)


def _dispatch_kernel(x_hbm, sl_hbm, w0x_hbm, w1x_hbm, xs_hbm, gv_hbm,
                     xrows_v, w0_v, w1_v, idx0_v, idx1_v, sem):
    wid = lax.axis_index("s") * 2 + lax.axis_index("c")
    base = wid * TPW
    pltpu.sync_copy(x_hbm.at[pl.ds(base, TPW)], xrows_v)
    pltpu.sync_copy(w0x_hbm.at[pl.ds(base, TPW)], w0_v)
    pltpu.sync_copy(w1x_hbm.at[pl.ds(base, TPW)], w1_v)
    pltpu.sync_copy(sl_hbm.at[0, pl.ds(base, TPW)], idx0_v)
    pltpu.sync_copy(sl_hbm.at[1, pl.ds(base, TPW)], idx1_v)
    c0 = pltpu.make_async_copy(xrows_v, xs_hbm.at[idx0_v], sem)
    c1 = pltpu.make_async_copy(xrows_v, xs_hbm.at[idx1_v], sem)
    g0 = pltpu.make_async_copy(w0_v, gv_hbm.at[idx0_v], sem)
    g1 = pltpu.make_async_copy(w1_v, gv_hbm.at[idx1_v], sem)
    c0.start()
    c1.start()
    g0.start()
    g1.start()
    c0.wait()
    c1.wait()
    g0.wait()
    g1.wait()


@jax.jit
def _dispatch(x, slT, w0x, w1x):
    mesh = plsc.VectorSubcoreMesh(core_axis_name="c", subcore_axis_name="s")
    f = functools.partial(
        pl.kernel,
        mesh=mesh,
        out_type=[
            jax.ShapeDtypeStruct((P, H), jnp.float32),
            jax.ShapeDtypeStruct((P, GW), jnp.float32),
        ],
        scratch_types=[
            pltpu.VMEM((TPW, H), jnp.float32),
            pltpu.VMEM((TPW, GW), jnp.float32),
            pltpu.VMEM((TPW, GW), jnp.float32),
            pltpu.VMEM((TPW,), jnp.int32),
            pltpu.VMEM((TPW,), jnp.int32),
            pltpu.SemaphoreType.DMA,
        ],
    )(_dispatch_kernel)
    return f(x, slT, w0x, w1x)


def _gmm_body(eid_ref, act_ref, x_ref, w1_ref, b1_ref, w2_ref, b2_ref,
              gv_ref, hb_ref, out_ref):
    i = pl.program_id(0)

    @pl.when(act_ref[i] == 1)
    def _compute():
        xt = x_ref[...].astype(jnp.bfloat16)
        y = jnp.zeros((TM, H), jnp.float32)
        for fc in range(FF // FC):
            w1c = w1_ref[0, pl.ds(fc * FC, FC), :].astype(jnp.bfloat16)
            h = lax.dot_general(
                xt, w1c, (((1,), (1,)), ((), ())),
                preferred_element_type=jnp.float32)
            h = h + b1_ref[0, :, pl.ds(fc * FC, FC)]
            h = (0.5 * h * (1.0 + lax.erf(h * 0.7071067811865476))).astype(
                jnp.bfloat16)
            w2c = w2_ref[0, :, pl.ds(fc * FC, FC)].astype(jnp.bfloat16)
            y = y + lax.dot_general(
                h, w2c, (((1,), (1,)), ((), ())),
                preferred_element_type=jnp.float32)
        g = gv_ref[:, 0:1]
        out_ref[...] = (y + b2_ref[0]) * g + hb_ref[...]


@jax.jit
def _gmm(eid, act, xs, W1, b1, W2, b2, gv, halfbias):
    grid_spec = pltpu.PrefetchScalarGridSpec(
        num_scalar_prefetch=2,
        grid=(NT,),
        in_specs=[
            pl.BlockSpec((TM, H), lambda i, eid, act: (act[i] * i, 0)),
            pl.BlockSpec((1, FF, H), lambda i, eid, act: (eid[i], 0, 0)),
            pl.BlockSpec((1, 1, FF), lambda i, eid, act: (eid[i], 0, 0)),
            pl.BlockSpec((1, H, FF), lambda i, eid, act: (eid[i], 0, 0)),
            pl.BlockSpec((1, 1, H), lambda i, eid, act: (eid[i], 0, 0)),
            pl.BlockSpec((TM, GW), lambda i, eid, act: (act[i] * i, 0)),
            pl.BlockSpec((1, H), lambda i, eid, act: (0, 0)),
        ],
        out_specs=pl.BlockSpec(
            (TM, H),
            lambda i, eid, act: (act[i] * i + (1 - act[i]) * (NT - 1), 0)),
    )
    return pl.pallas_call(
        _gmm_body,
        grid_spec=grid_spec,
        out_shape=jax.ShapeDtypeStruct((P, H), jnp.float32),
    )(eid, act, xs, W1, b1.reshape(E, 1, FF), W2, b2.reshape(E, 1, H), gv,
      halfbias)


def _combine_kernel(yg_hbm, sl_hbm, out_hbm, buf0_v, buf1_v,
                    idx0_v, idx1_v, sem):
    wid = lax.axis_index("s") * 2 + lax.axis_index("c")
    base = wid * TPW
    pltpu.sync_copy(sl_hbm.at[0, pl.ds(base, TPW)], idx0_v)
    pltpu.sync_copy(sl_hbm.at[1, pl.ds(base, TPW)], idx1_v)
    c0 = pltpu.make_async_copy(yg_hbm.at[idx0_v], buf0_v, sem)
    c1 = pltpu.make_async_copy(yg_hbm.at[idx1_v], buf1_v, sem)
    c0.start()
    c1.start()
    c0.wait()
    c1.wait()

    def row(r, carry):
        for c in range(H // 16):
            sl = pl.ds(c * 16, 16)
            buf0_v[r, sl] = buf0_v[r, sl] + buf1_v[r, sl]
        return carry

    lax.fori_loop(0, TPW, row, 0)
    pltpu.sync_copy(buf0_v, out_hbm.at[pl.ds(base, TPW)])


@jax.jit
def _combine(yg, slT):
    mesh = plsc.VectorSubcoreMesh(core_axis_name="c", subcore_axis_name="s")
    f = functools.partial(
        pl.kernel,
        mesh=mesh,
        out_type=jax.ShapeDtypeStruct((T, H), jnp.float32),
        scratch_types=[
            pltpu.VMEM((TPW, H), jnp.float32),
            pltpu.VMEM((TPW, H), jnp.float32),
            pltpu.VMEM((TPW,), jnp.int32),
            pltpu.VMEM((TPW,), jnp.int32),
            pltpu.SemaphoreType.DMA,
        ],
    )(_combine_kernel)
    return f(yg, slT)


def kernel(hidden_states, Wr, W1, b1, W2, b2, bias):
    input_shape = hidden_states.shape
    x = hidden_states.reshape(-1, input_shape[-1])
    logits, w0x, w1x, slot01, eid, act = _router(x, Wr)
    slT = slot01.T
    xs, gv = _dispatch(x, slT, w0x, w1x)
    yg = _gmm(eid.reshape(NT), act.reshape(NT), xs, W1, b1, W2, b2, gv,
              0.5 * bias.reshape(1, H))
    out = _combine(yg, slT)
    return out.reshape(input_shape), logits
